# baseline (speedup 1.0000x reference)
import functools

import jax
import jax.numpy as jnp
from jax import lax
from jax.experimental import pallas as pl
from jax.experimental.pallas import tpu as pltpu
from jax.experimental.pallas import tpu_sc as plsc

NC = 2
NS = 16
NW = NC * NS
LANES = 16
CHUNK = 128


def _make_kernel(batch, embed_dim):
    assert batch % (8 * NW) == 0
    bpw = batch // NW
    n_chunks = bpw // CHUNK
    n_groups = bpw // LANES

    mesh = plsc.VectorSubcoreMesh(core_axis_name="c", subcore_axis_name="s")

    @functools.partial(
        pl.kernel,
        out_type=jax.ShapeDtypeStruct((batch,), jnp.float32),
        mesh=mesh,
        compiler_params=pltpu.CompilerParams(
            needs_layout_passes=False, use_tc_tiling_on_sc=False),
        scratch_types=[
            pltpu.VMEM((n_chunks, CHUNK), jnp.int32),
            pltpu.VMEM((n_chunks, CHUNK), jnp.int32),
            pltpu.VMEM((bpw, embed_dim), jnp.float32),
            pltpu.VMEM((bpw, embed_dim), jnp.float32),
            pltpu.VMEM((bpw,), jnp.float32),
            pltpu.VMEM((bpw,), jnp.float32),
            pltpu.VMEM((bpw,), jnp.float32),
            pltpu.SemaphoreType.DMA,
        ],
    )
    def k(uid_hbm, iid_hbm, ut_hbm, ubt_hbm, it_hbm, ibt_hbm, out_hbm,
          uidx, iidx, urows, irows, ubias, ibias, outv, sem):
        wid = lax.axis_index("s") * NC + lax.axis_index("c")
        base = wid * bpw

        for j in range(n_chunks):
            pltpu.sync_copy(uid_hbm.at[pl.ds(base + j * CHUNK, CHUNK)], uidx.at[j])
            pltpu.sync_copy(iid_hbm.at[pl.ds(base + j * CHUNK, CHUNK)], iidx.at[j])

        copies = []
        for j in range(n_chunks):
            sl = pl.ds(j * CHUNK, CHUNK)
            copies.append(pltpu.async_copy(ut_hbm.at[uidx.at[j]], urows.at[sl], sem))
            copies.append(pltpu.async_copy(it_hbm.at[iidx.at[j]], irows.at[sl], sem))
            copies.append(pltpu.async_copy(ubt_hbm.at[uidx.at[j]], ubias.at[sl], sem))
            copies.append(pltpu.async_copy(ibt_hbm.at[iidx.at[j]], ibias.at[sl], sem))
        for cp in copies:
            cp.wait()

        lane = lax.iota(jnp.int32, LANES)

        def group(g, carry):
            b = g * LANES + lane
            acc = plsc.load_gather(ubias, [b]) + plsc.load_gather(ibias, [b])
            for d in range(embed_dim):
                dv = jnp.full((LANES,), d, jnp.int32)
                acc = acc + plsc.load_gather(urows, [b, dv]) * plsc.load_gather(irows, [b, dv])
            plsc.store_scatter(outv, [b], acc)
            return carry

        lax.fori_loop(0, n_groups, group, 0)
        pltpu.sync_copy(outv, out_hbm.at[pl.ds(base, bpw)])

    return k


def kernel(user_id, item_id, user_table, user_bias_table, item_table, item_bias_table):
    batch = user_id.shape[0]
    embed_dim = user_table.shape[1]
    k = _make_kernel(batch, embed_dim)
    return k(user_id.astype(jnp.int32), item_id.astype(jnp.int32),
             user_table, user_bias_table.reshape(-1),
             item_table, item_bias_table.reshape(-1))


# speedup vs baseline: 2.3039x; 2.3039x over previous
import functools

import jax
import jax.numpy as jnp
from jax import lax
from jax.experimental import pallas as pl
from jax.experimental.pallas import tpu as pltpu
from jax.experimental.pallas import tpu_sc as plsc

NC = 2
NS = 16
NW = NC * NS
LANES = 16
SUB = 8
GC = 128
IDX_CHUNK = 128


def _make_kernel(batch, embed_dim):
    assert batch % (8 * NW) == 0
    bpw = batch // NW
    n_idx_chunks = bpw // IDX_CHUNK
    n_gchunks = bpw // GC
    n_groups = GC // LANES

    mesh = plsc.VectorSubcoreMesh(core_axis_name="c", subcore_axis_name="s")

    @functools.partial(
        pl.kernel,
        out_type=jax.ShapeDtypeStruct((batch,), jnp.float32),
        mesh=mesh,
        compiler_params=pltpu.CompilerParams(
            needs_layout_passes=False, use_tc_tiling_on_sc=True),
        scratch_types=[
            pltpu.VMEM((bpw,), jnp.int32),
            pltpu.VMEM((bpw,), jnp.int32),
            pltpu.VMEM((GC, embed_dim), jnp.float32),
            pltpu.VMEM((GC, embed_dim), jnp.float32),
            pltpu.VMEM((bpw,), jnp.float32),
            pltpu.VMEM((bpw,), jnp.float32),
            pltpu.VMEM((bpw,), jnp.float32),
            pltpu.SemaphoreType.DMA,
        ],
    )
    def k(uid_hbm, iid_hbm, ut_hbm, ubt_hbm, it_hbm, ibt_hbm, out_hbm,
          uidx, iidx, urows, irows, ubias, ibias, outv, sem):
        wid = lax.axis_index("s") * NC + lax.axis_index("c")
        base = wid * bpw

        bias_copies = []
        for j in range(n_idx_chunks):
            sl = pl.ds(j * IDX_CHUNK, IDX_CHUNK)
            pltpu.sync_copy(uid_hbm.at[pl.ds(base + j * IDX_CHUNK, IDX_CHUNK)],
                            uidx.at[sl])
            pltpu.sync_copy(iid_hbm.at[pl.ds(base + j * IDX_CHUNK, IDX_CHUNK)],
                            iidx.at[sl])
            bias_copies.append(
                pltpu.async_copy(ubt_hbm.at[uidx.at[sl]], ubias.at[sl], sem))
            bias_copies.append(
                pltpu.async_copy(ibt_hbm.at[iidx.at[sl]], ibias.at[sl], sem))

        lane = lax.iota(jnp.int32, LANES)

        def gchunk(j, carry):
            def fire(g, c):
                uvec = uidx[pl.ds(j * GC + g * LANES, LANES)]
                ivec = iidx[pl.ds(j * GC + g * LANES, LANES)]
                for i in range(LANES):
                    uid = uvec[i]
                    iid = ivec[i]
                    e = g * LANES + i
                    pltpu.async_copy(ut_hbm.at[uid >> 3, uid & 7, :],
                                     urows.at[e], sem)
                    pltpu.async_copy(it_hbm.at[iid >> 3, iid & 7, :],
                                     irows.at[e], sem)
                return c
            lax.fori_loop(0, n_groups, fire, 0)

            def drain(e, c):
                pltpu.make_async_copy(ut_hbm.at[0, 0, :], urows.at[e], sem).wait()
                pltpu.make_async_copy(it_hbm.at[0, 0, :], irows.at[e], sem).wait()
                return c
            lax.fori_loop(0, GC, drain, 0)

            for g in range(n_groups):
                el = g * LANES + lane
                acc = jnp.zeros((LANES,), jnp.float32)
                for d in range(embed_dim):
                    dv = jnp.full((LANES,), d, jnp.int32)
                    u = plsc.load_gather(urows, [el, dv])
                    v = plsc.load_gather(irows, [el, dv])
                    acc = acc + u * v
                plsc.store_scatter(outv, [jnp.full((LANES,), j * GC, jnp.int32) + el], acc)
            return carry
        lax.fori_loop(0, n_gchunks, gchunk, 0)

        for cp in bias_copies:
            cp.wait()

        def biasadd(g, carry):
            sl = pl.ds(g * LANES, LANES)
            outv[sl] = outv[sl] + ubias[sl] + ibias[sl]
            return carry
        lax.fori_loop(0, bpw // LANES, biasadd, 0)

        pltpu.sync_copy(outv, out_hbm.at[pl.ds(base, bpw)])

    return k


def kernel(user_id, item_id, user_table, user_bias_table, item_table, item_bias_table):
    batch = user_id.shape[0]
    num_rows, embed_dim = user_table.shape
    k = _make_kernel(batch, embed_dim)
    return k(user_id.astype(jnp.int32), item_id.astype(jnp.int32),
             user_table.reshape(num_rows // SUB, SUB, embed_dim),
             user_bias_table.reshape(-1),
             item_table.reshape(num_rows // SUB, SUB, embed_dim),
             item_bias_table.reshape(-1))
